# baseline (speedup 1.0000x reference)
import functools

import jax
import jax.numpy as jnp
from jax import lax
from jax.experimental import pallas as pl
from jax.experimental.pallas import tpu as pltpu
from jax.experimental.pallas import tpu_sc as plsc

BATCH = 32
POS = 1024
DIM = 768

NUM_CORES = 2
NUM_SUBCORES = 16
NW = NUM_CORES * NUM_SUBCORES
PPW = POS // NW
LANES = 16
NBUF = 4
NROUND = BATCH // NBUF

_MESH = plsc.VectorSubcoreMesh(
    core_axis_name="c", subcore_axis_name="s",
    num_cores=NUM_CORES, num_subcores=NUM_SUBCORES)


@functools.partial(
    pl.kernel,
    out_type=jax.ShapeDtypeStruct((BATCH, POS, DIM), jnp.float32),
    mesh=_MESH,
    scratch_types=[
        pltpu.VMEM((PPW, DIM), jnp.float32),
        pltpu.VMEM((PPW, DIM), jnp.float32),
        pltpu.VMEM((PPW, DIM), jnp.float32),
        pltpu.VMEM((PPW, DIM), jnp.float32),
        pltpu.VMEM((PPW, DIM), jnp.float32),
        pltpu.SemaphoreType.DMA,
        pltpu.SemaphoreType.DMA,
        pltpu.SemaphoreType.DMA,
        pltpu.SemaphoreType.DMA,
        pltpu.SemaphoreType.DMA,
        pltpu.SemaphoreType.DMA,
        pltpu.SemaphoreType.DMA,
        pltpu.SemaphoreType.DMA,
    ],
)
def _pos_add_sc(x_hbm, pos_hbm, out_hbm, pos_v, b0, b1, b2, b3,
                si0, si1, si2, si3, so0, so1, so2, so3):
    w = lax.axis_index("s") * NUM_CORES + lax.axis_index("c")
    rows = pl.ds(w * PPW, PPW)

    bufs = (b0, b1, b2, b3)
    sin = (si0, si1, si2, si3)
    sout = (so0, so1, so2, so3)

    pltpu.sync_copy(pos_hbm.at[rows], pos_v)
    pltpu.async_copy(x_hbm.at[0, rows], b0, si0)
    pltpu.async_copy(x_hbm.at[1, rows], b1, si1)

    def round_body(j, carry):
        for s in range(NBUF):
            b = NBUF * j + s
            f = (s + 2) % NBUF
            @pl.when(b >= 2)
            def _retire(f=f, b=b):
                pltpu.make_async_copy(bufs[f], out_hbm.at[b - 2, rows],
                                      sout[f]).wait()

            @pl.when(b + 2 < BATCH)
            def _prefetch(f=f, b=b):
                pltpu.async_copy(x_hbm.at[b + 2, rows], bufs[f], sin[f])

            pltpu.make_async_copy(x_hbm.at[b, rows], bufs[s], sin[s]).wait()

            @plsc.parallel_loop(0, PPW)
            def _add(i, s=s):
                for c in range(0, DIM, LANES):
                    sl = pl.ds(c, LANES)
                    plsc.addupdate(bufs[s].at[i, sl], pos_v[i, sl])

            pltpu.async_copy(bufs[s], out_hbm.at[b, rows], sout[s])
        return carry

    lax.fori_loop(0, NROUND, round_body, 0)

    pltpu.make_async_copy(b2, out_hbm.at[BATCH - 2, rows], so2).wait()
    pltpu.make_async_copy(b3, out_hbm.at[BATCH - 1, rows], so3).wait()


def kernel(inputs, pos_table):
    return _pos_add_sc(inputs, pos_table)


# speedup vs baseline: 1.0179x; 1.0179x over previous
import functools

import jax
import jax.numpy as jnp
from jax import lax
from jax.experimental import pallas as pl
from jax.experimental.pallas import tpu as pltpu
from jax.experimental.pallas import tpu_sc as plsc

BATCH = 32
POS = 1024
DIM = 768

NUM_CORES = 2
NUM_SUBCORES = 16
NW = NUM_CORES * NUM_SUBCORES
PPW = POS // NW
LANES = 16
NBUF = 4
NROUND = BATCH // NBUF

_MESH = plsc.VectorSubcoreMesh(
    core_axis_name="c", subcore_axis_name="s",
    num_cores=NUM_CORES, num_subcores=NUM_SUBCORES)


@functools.partial(
    pl.kernel,
    out_type=jax.ShapeDtypeStruct((BATCH, POS, DIM), jnp.float32),
    mesh=_MESH,
    scratch_types=[
        pltpu.VMEM((PPW, DIM), jnp.float32),
        pltpu.VMEM((PPW, DIM), jnp.float32),
        pltpu.VMEM((PPW, DIM), jnp.float32),
        pltpu.VMEM((PPW, DIM), jnp.float32),
        pltpu.VMEM((PPW, DIM), jnp.float32),
        pltpu.SemaphoreType.DMA,
        pltpu.SemaphoreType.DMA,
        pltpu.SemaphoreType.DMA,
        pltpu.SemaphoreType.DMA,
        pltpu.SemaphoreType.DMA,
        pltpu.SemaphoreType.DMA,
        pltpu.SemaphoreType.DMA,
        pltpu.SemaphoreType.DMA,
    ],
)
def _pos_add_sc(x_hbm, pos_hbm, out_hbm, pos_v, b0, b1, b2, b3,
                si0, si1, si2, si3, so0, so1, so2, so3):
    w = lax.axis_index("s") * NUM_CORES + lax.axis_index("c")
    rows = pl.ds(w * PPW, PPW)

    bufs = (b0, b1, b2, b3)
    sin = (si0, si1, si2, si3)
    sout = (so0, so1, so2, so3)

    pltpu.async_copy(x_hbm.at[0, rows], b0, si0)
    pltpu.async_copy(x_hbm.at[1, rows], b1, si1)
    pltpu.sync_copy(pos_hbm.at[rows], pos_v)

    def round_body(j, carry):
        for s in range(NBUF):
            b = NBUF * j + s
            f = (s + 2) % NBUF
            @pl.when(b >= 2)
            def _retire(f=f, b=b):
                pltpu.make_async_copy(bufs[f], out_hbm.at[b - 2, rows],
                                      sout[f]).wait()

            @pl.when(b + 2 < BATCH)
            def _prefetch(f=f, b=b):
                pltpu.async_copy(x_hbm.at[b + 2, rows], bufs[f], sin[f])

            pltpu.make_async_copy(x_hbm.at[b, rows], bufs[s], sin[s]).wait()

            @plsc.parallel_loop(0, PPW)
            def _add(i, s=s):
                for c in range(0, DIM, LANES):
                    sl = pl.ds(c, LANES)
                    plsc.addupdate(bufs[s].at[i, sl], pos_v[i, sl])

            pltpu.async_copy(bufs[s], out_hbm.at[b, rows], sout[s])
        return carry

    lax.fori_loop(0, NROUND, round_body, 0)

    pltpu.make_async_copy(b2, out_hbm.at[BATCH - 2, rows], so2).wait()
    pltpu.make_async_copy(b3, out_hbm.at[BATCH - 1, rows], so3).wait()


def kernel(inputs, pos_table):
    return _pos_add_sc(inputs, pos_table)
